# baseline (speedup 1.0000x reference)
import jax
import jax.numpy as jnp
from jax import lax
from jax.experimental import pallas as pl
from jax.experimental.pallas import tpu as pltpu

_HIDDEN = 2048
_Q_SIZE = 2048
_KV_SIZE = 512
_OUT_SIZE = _Q_SIZE + 2 * _KV_SIZE
_R = 16
_SCALING = 2.0

_BM = 1024
_WCH = 256


def _body(x_ref, w_hbm, b_ref, a_ref, o_ref, weff_ref, wtmp_ref, sem):
    @pl.when(pl.program_id(0) == 0)
    def _merge():
        def chunk(i, carry):
            cp = pltpu.make_async_copy(
                w_hbm.at[pl.ds(i * _WCH, _WCH), :], wtmp_ref, sem)
            cp.start()
            cp.wait()
            ba = jax.lax.dot_general(
                b_ref[:, pl.ds(i * _WCH, _WCH)], a_ref[...],
                (((0,), (0,)), ((), ())),
                preferred_element_type=jnp.float32)
            weff_ref[pl.ds(i * _WCH, _WCH), :] = (
                wtmp_ref[...] + _SCALING * ba
            ).astype(jnp.bfloat16)
            return carry

        lax.fori_loop(0, _OUT_SIZE // _WCH, chunk, 0)

    o_ref[...] = jax.lax.dot_general(
        x_ref[...].astype(jnp.bfloat16), weff_ref[...],
        (((1,), (1,)), ((), ())),
        preferred_element_type=jnp.float32)


def kernel(x, weight, lora_A, lora_B_q, lora_B_k, lora_B_v):
    orig_shape = x.shape
    x_flat = x.reshape(-1, x.shape[-1])
    m_total = x_flat.shape[0]

    b_exp = jnp.zeros((3 * _R, _OUT_SIZE), jnp.bfloat16)
    b_exp = b_exp.at[:_R, :_Q_SIZE].set(lora_B_q[0].T.astype(jnp.bfloat16))
    b_exp = b_exp.at[_R:2 * _R, _Q_SIZE:_Q_SIZE + _KV_SIZE].set(
        lora_B_k[0].T.astype(jnp.bfloat16))
    b_exp = b_exp.at[2 * _R:, _Q_SIZE + _KV_SIZE:].set(
        lora_B_v[0].T.astype(jnp.bfloat16))
    a_stack = lora_A[0].reshape(3 * _R, _HIDDEN).astype(jnp.bfloat16)

    out = pl.pallas_call(
        _body,
        grid=(m_total // _BM,),
        in_specs=[
            pl.BlockSpec((_BM, _HIDDEN), lambda m: (m, 0)),
            pl.BlockSpec(memory_space=pl.ANY),
            pl.BlockSpec((3 * _R, _OUT_SIZE), lambda m: (0, 0)),
            pl.BlockSpec((3 * _R, _HIDDEN), lambda m: (0, 0)),
        ],
        out_specs=pl.BlockSpec((_BM, _OUT_SIZE), lambda m: (m, 0)),
        out_shape=jax.ShapeDtypeStruct((m_total, _OUT_SIZE), jnp.float32),
        scratch_shapes=[
            pltpu.VMEM((_OUT_SIZE, _HIDDEN), jnp.bfloat16),
            pltpu.VMEM((_WCH, _HIDDEN), jnp.float32),
            pltpu.SemaphoreType.DMA,
        ],
        compiler_params=pltpu.CompilerParams(
            dimension_semantics=("arbitrary",)),
    )(x_flat, weight, b_exp, a_stack)
    return out.reshape(*orig_shape[:-1], _OUT_SIZE)


# speedup vs baseline: 1.0220x; 1.0220x over previous
import jax
import jax.numpy as jnp
from jax import lax
from jax.experimental import pallas as pl
from jax.experimental.pallas import tpu as pltpu

_HIDDEN = 2048
_Q_SIZE = 2048
_KV_SIZE = 512
_OUT_SIZE = _Q_SIZE + 2 * _KV_SIZE
_R = 16
_SCALING = 2.0

_BM = 256
_WCH = 512


def _body(x_ref, w_hbm, b_ref, a_ref, o_ref, weff_ref, wtmp_ref, sem):
    @pl.when(pl.program_id(0) == 0)
    def _merge():
        def chunk(i, carry):
            cp = pltpu.make_async_copy(
                w_hbm.at[pl.ds(i * _WCH, _WCH), :], wtmp_ref, sem)
            cp.start()
            cp.wait()
            ba = jax.lax.dot_general(
                b_ref[:, pl.ds(i * _WCH, _WCH)], a_ref[...],
                (((0,), (0,)), ((), ())),
                preferred_element_type=jnp.float32)
            weff_ref[pl.ds(i * _WCH, _WCH), :] = (
                wtmp_ref[...] + _SCALING * ba
            ).astype(jnp.bfloat16)
            return carry

        lax.fori_loop(0, _OUT_SIZE // _WCH, chunk, 0)

    o_ref[...] = jax.lax.dot_general(
        x_ref[...].astype(jnp.bfloat16), weff_ref[...],
        (((1,), (1,)), ((), ())),
        preferred_element_type=jnp.float32)


def kernel(x, weight, lora_A, lora_B_q, lora_B_k, lora_B_v):
    orig_shape = x.shape
    x_flat = x.reshape(-1, x.shape[-1])
    m_total = x_flat.shape[0]

    b_exp = jnp.zeros((3 * _R, _OUT_SIZE), jnp.bfloat16)
    b_exp = b_exp.at[:_R, :_Q_SIZE].set(lora_B_q[0].T.astype(jnp.bfloat16))
    b_exp = b_exp.at[_R:2 * _R, _Q_SIZE:_Q_SIZE + _KV_SIZE].set(
        lora_B_k[0].T.astype(jnp.bfloat16))
    b_exp = b_exp.at[2 * _R:, _Q_SIZE + _KV_SIZE:].set(
        lora_B_v[0].T.astype(jnp.bfloat16))
    a_stack = lora_A[0].reshape(3 * _R, _HIDDEN).astype(jnp.bfloat16)

    out = pl.pallas_call(
        _body,
        grid=(m_total // _BM,),
        in_specs=[
            pl.BlockSpec((_BM, _HIDDEN), lambda m: (m, 0)),
            pl.BlockSpec(memory_space=pl.ANY),
            pl.BlockSpec((3 * _R, _OUT_SIZE), lambda m: (0, 0)),
            pl.BlockSpec((3 * _R, _HIDDEN), lambda m: (0, 0)),
        ],
        out_specs=pl.BlockSpec((_BM, _OUT_SIZE), lambda m: (m, 0)),
        out_shape=jax.ShapeDtypeStruct((m_total, _OUT_SIZE), jnp.float32),
        scratch_shapes=[
            pltpu.VMEM((_OUT_SIZE, _HIDDEN), jnp.bfloat16),
            pltpu.VMEM((_WCH, _HIDDEN), jnp.float32),
            pltpu.SemaphoreType.DMA,
        ],
        compiler_params=pltpu.CompilerParams(
            dimension_semantics=("arbitrary",)),
    )(x_flat, weight, b_exp, a_stack)
    return out.reshape(*orig_shape[:-1], _OUT_SIZE)


# speedup vs baseline: 1.0798x; 1.0566x over previous
import jax
import jax.numpy as jnp
from jax import lax
from jax.experimental import pallas as pl
from jax.experimental.pallas import tpu as pltpu

_HIDDEN = 2048
_Q_SIZE = 2048
_KV_SIZE = 512
_OUT_SIZE = _Q_SIZE + 2 * _KV_SIZE
_R = 16
_SCALING = 2.0

_BM = 256
_WCH = 512


def _body(x_ref, w_hbm, b_ref, a_ref, o_ref, weff_ref, wtmp_ref, sem):
    @pl.when(pl.program_id(0) == 0)
    def _merge():
        nch = _OUT_SIZE // _WCH

        def start(i, slot):
            pltpu.make_async_copy(
                w_hbm.at[pl.ds(i * _WCH, _WCH), :],
                wtmp_ref.at[slot], sem.at[slot]).start()

        start(0, 0)
        for i in range(nch):
            if i + 1 < nch:
                start(i + 1, (i + 1) % 2)
            pltpu.make_async_copy(
                w_hbm.at[pl.ds(i * _WCH, _WCH), :],
                wtmp_ref.at[i % 2], sem.at[i % 2]).wait()
            ba = jax.lax.dot_general(
                b_ref[:, pl.ds(i * _WCH, _WCH)], a_ref[...],
                (((0,), (0,)), ((), ())),
                preferred_element_type=jnp.float32)
            weff_ref[pl.ds(i * _WCH, _WCH), :] = (
                wtmp_ref[i % 2] + _SCALING * ba
            ).astype(jnp.bfloat16)

    o_ref[...] = jax.lax.dot_general(
        x_ref[...].astype(jnp.bfloat16), weff_ref[...],
        (((1,), (1,)), ((), ())),
        preferred_element_type=jnp.float32)


def kernel(x, weight, lora_A, lora_B_q, lora_B_k, lora_B_v):
    orig_shape = x.shape
    x_flat = x.reshape(-1, x.shape[-1])
    m_total = x_flat.shape[0]

    b_exp = jnp.zeros((3 * _R, _OUT_SIZE), jnp.bfloat16)
    b_exp = b_exp.at[:_R, :_Q_SIZE].set(lora_B_q[0].T.astype(jnp.bfloat16))
    b_exp = b_exp.at[_R:2 * _R, _Q_SIZE:_Q_SIZE + _KV_SIZE].set(
        lora_B_k[0].T.astype(jnp.bfloat16))
    b_exp = b_exp.at[2 * _R:, _Q_SIZE + _KV_SIZE:].set(
        lora_B_v[0].T.astype(jnp.bfloat16))
    a_stack = lora_A[0].reshape(3 * _R, _HIDDEN).astype(jnp.bfloat16)

    out = pl.pallas_call(
        _body,
        grid=(m_total // _BM,),
        in_specs=[
            pl.BlockSpec((_BM, _HIDDEN), lambda m: (m, 0)),
            pl.BlockSpec(memory_space=pl.ANY),
            pl.BlockSpec((3 * _R, _OUT_SIZE), lambda m: (0, 0)),
            pl.BlockSpec((3 * _R, _HIDDEN), lambda m: (0, 0)),
        ],
        out_specs=pl.BlockSpec((_BM, _OUT_SIZE), lambda m: (m, 0)),
        out_shape=jax.ShapeDtypeStruct((m_total, _OUT_SIZE), jnp.float32),
        scratch_shapes=[
            pltpu.VMEM((_OUT_SIZE, _HIDDEN), jnp.bfloat16),
            pltpu.VMEM((2, _WCH, _HIDDEN), jnp.float32),
            pltpu.SemaphoreType.DMA((2,)),
        ],
        compiler_params=pltpu.CompilerParams(
            dimension_semantics=("arbitrary",)),
    )(x_flat, weight, b_exp, a_stack)
    return out.reshape(*orig_shape[:-1], _OUT_SIZE)


# speedup vs baseline: 1.1220x; 1.0391x over previous
import jax
import jax.numpy as jnp
from jax import lax
from jax.experimental import pallas as pl
from jax.experimental.pallas import tpu as pltpu

_HIDDEN = 2048
_Q_SIZE = 2048
_KV_SIZE = 512
_OUT_SIZE = _Q_SIZE + 2 * _KV_SIZE
_R = 16
_SCALING = 2.0

_BM = 512
_WCH = 1024


def _body(x_ref, w_hbm, b_ref, a_ref, o_ref, weff_ref, wtmp_ref, sem):
    @pl.when(pl.program_id(0) == 0)
    def _merge():
        nch = _OUT_SIZE // _WCH

        def start(i, slot):
            pltpu.make_async_copy(
                w_hbm.at[pl.ds(i * _WCH, _WCH), :],
                wtmp_ref.at[slot], sem.at[slot]).start()

        start(0, 0)
        for i in range(nch):
            if i + 1 < nch:
                start(i + 1, (i + 1) % 2)
            pltpu.make_async_copy(
                w_hbm.at[pl.ds(i * _WCH, _WCH), :],
                wtmp_ref.at[i % 2], sem.at[i % 2]).wait()
            ba = jax.lax.dot_general(
                b_ref[:, pl.ds(i * _WCH, _WCH)], a_ref[...],
                (((0,), (0,)), ((), ())),
                preferred_element_type=jnp.float32)
            weff_ref[pl.ds(i * _WCH, _WCH), :] = (
                wtmp_ref[i % 2] + _SCALING * ba
            ).astype(jnp.bfloat16)

    o_ref[...] = jax.lax.dot_general(
        x_ref[...].astype(jnp.bfloat16), weff_ref[...],
        (((1,), (1,)), ((), ())),
        preferred_element_type=jnp.float32)


def kernel(x, weight, lora_A, lora_B_q, lora_B_k, lora_B_v):
    orig_shape = x.shape
    x_flat = x.reshape(-1, x.shape[-1])
    m_total = x_flat.shape[0]

    b_exp = jnp.zeros((3 * _R, _OUT_SIZE), jnp.bfloat16)
    b_exp = b_exp.at[:_R, :_Q_SIZE].set(lora_B_q[0].T.astype(jnp.bfloat16))
    b_exp = b_exp.at[_R:2 * _R, _Q_SIZE:_Q_SIZE + _KV_SIZE].set(
        lora_B_k[0].T.astype(jnp.bfloat16))
    b_exp = b_exp.at[2 * _R:, _Q_SIZE + _KV_SIZE:].set(
        lora_B_v[0].T.astype(jnp.bfloat16))
    a_stack = lora_A[0].reshape(3 * _R, _HIDDEN).astype(jnp.bfloat16)

    out = pl.pallas_call(
        _body,
        grid=(m_total // _BM,),
        in_specs=[
            pl.BlockSpec((_BM, _HIDDEN), lambda m: (m, 0)),
            pl.BlockSpec(memory_space=pl.ANY),
            pl.BlockSpec((3 * _R, _OUT_SIZE), lambda m: (0, 0)),
            pl.BlockSpec((3 * _R, _HIDDEN), lambda m: (0, 0)),
        ],
        out_specs=pl.BlockSpec((_BM, _OUT_SIZE), lambda m: (m, 0)),
        out_shape=jax.ShapeDtypeStruct((m_total, _OUT_SIZE), jnp.float32),
        scratch_shapes=[
            pltpu.VMEM((_OUT_SIZE, _HIDDEN), jnp.bfloat16),
            pltpu.VMEM((2, _WCH, _HIDDEN), jnp.float32),
            pltpu.SemaphoreType.DMA((2,)),
        ],
        compiler_params=pltpu.CompilerParams(
            dimension_semantics=("arbitrary",)),
    )(x_flat, weight, b_exp, a_stack)
    return out.reshape(*orig_shape[:-1], _OUT_SIZE)
